# baseline (speedup 1.0000x reference)
import functools

import jax
import jax.numpy as jnp
from jax import lax
from jax.experimental import pallas as pl
from jax.experimental.pallas import tpu as pltpu
from jax.experimental.pallas import tpu_sc as plsc


def kernel(item_ids, item_biases):
    batch = item_ids.shape[0]

    info = plsc.get_sparse_core_info()
    num_workers = info.num_cores * info.num_subcores
    bpw = batch // num_workers

    table = item_biases.reshape(-1)

    mesh = plsc.VectorSubcoreMesh(core_axis_name="c", subcore_axis_name="s")

    @functools.partial(
        pl.kernel,
        out_type=jax.ShapeDtypeStruct((batch,), jnp.float32),
        mesh=mesh,
        scratch_types=[
            pltpu.VMEM((bpw,), jnp.int32),
            pltpu.VMEM((bpw,), jnp.float32),
            pltpu.SemaphoreType.DMA,
            pltpu.SemaphoreType.DMA,
            pltpu.SemaphoreType.DMA,
        ],
    )
    def gather(table_hbm, idx_hbm, out_hbm, idx_v, vals_v, s_idx, s_g, s_out):
        wid = lax.axis_index("s") * info.num_cores + lax.axis_index("c")
        base = wid * bpw
        half = bpw // 2
        stage = [
            pltpu.async_copy(
                idx_hbm.at[pl.ds(base + h * half, half)],
                idx_v.at[pl.ds(h * half, half)],
                s_idx,
            )
            for h in range(2)
        ]
        gathers = []
        for h in range(2):
            stage[h].wait()
            gathers.append(
                pltpu.async_copy(
                    table_hbm.at[idx_v.at[pl.ds(h * half, half)]],
                    vals_v.at[pl.ds(h * half, half)],
                    s_g,
                )
            )
        stores = []
        for h in range(2):
            gathers[h].wait()
            stores.append(
                pltpu.async_copy(
                    vals_v.at[pl.ds(h * half, half)],
                    out_hbm.at[pl.ds(base + h * half, half)],
                    s_out,
                )
            )
        for st in stores:
            st.wait()

    return gather(table, item_ids)


# speedup vs baseline: 1.0006x; 1.0006x over previous
import functools

import jax
import jax.numpy as jnp
from jax import lax
from jax.experimental import pallas as pl
from jax.experimental.pallas import tpu as pltpu
from jax.experimental.pallas import tpu_sc as plsc


def kernel(item_ids, item_biases):
    batch = item_ids.shape[0]

    info = plsc.get_sparse_core_info()
    num_workers = info.num_cores * info.num_subcores
    bpw = batch // num_workers

    table = item_biases.reshape(-1)

    mesh = plsc.VectorSubcoreMesh(core_axis_name="c", subcore_axis_name="s")

    @functools.partial(
        pl.kernel,
        out_type=jax.ShapeDtypeStruct((batch,), jnp.float32),
        mesh=mesh,
        scratch_types=[
            pltpu.VMEM((bpw,), jnp.int32),
            pltpu.VMEM((bpw,), jnp.float32),
            pltpu.SemaphoreType.DMA,
            pltpu.SemaphoreType.DMA,
            pltpu.SemaphoreType.DMA,
        ],
    )
    def gather(table_hbm, idx_hbm, out_hbm, idx_v, vals_v, s_idx, s_g, s_out):
        wid = lax.axis_index("s") * info.num_cores + lax.axis_index("c")
        base = wid * bpw
        nchunks = 4
        chunk = bpw // nchunks
        stage = [
            pltpu.async_copy(
                idx_hbm.at[pl.ds(base + h * chunk, chunk)],
                idx_v.at[pl.ds(h * chunk, chunk)],
                s_idx,
            )
            for h in range(nchunks)
        ]
        gathers = []
        for h in range(nchunks):
            stage[h].wait()
            gathers.append(
                pltpu.async_copy(
                    table_hbm.at[idx_v.at[pl.ds(h * chunk, chunk)]],
                    vals_v.at[pl.ds(h * chunk, chunk)],
                    s_g,
                )
            )
        stores = []
        for h in range(nchunks):
            gathers[h].wait()
            stores.append(
                pltpu.async_copy(
                    vals_v.at[pl.ds(h * chunk, chunk)],
                    out_hbm.at[pl.ds(base + h * chunk, chunk)],
                    s_out,
                )
            )
        for st in stores:
            st.wait()

    return gather(table, item_ids)
